# Initial kernel scaffold; baseline (speedup 1.0000x reference)
import jax
import jax.numpy as jnp
from jax.experimental import pallas as pl


def kernel(prediction, target, target_sizes):
    raise NotImplementedError("write your pallas kernel here")


# speedup vs baseline: 134.0208x; 134.0208x over previous
import functools

import jax
import jax.numpy as jnp
from jax import lax
from jax.experimental import pallas as pl
from jax.experimental.pallas import tpu as pltpu
from jax.experimental.pallas import tpu_sc as plsc

_NUM_CLASSES = 20
_SCALE = 16.0
_IGNORE_THRESH = 0.5
_BAD_CONF_WEIGHT = 1.25
_ANCHORS = [(25.0, 50.0), (50.0, 100.0), (100.0, 200.0), (200.0, 120.0),
            (320.0, 320.0)]

_B, _NA, _NH, _NW, _MAXT = 16, 5, 32, 32, 50
_CELLS = _NH * _NW
_GRID = _NA * _CELLS
_NFIELD = 16
_NSEC = 7
_REC_FLAT = _NFIELD * _MAXT
_COMB = _NSEC * _GRID


def _records_body(tgt_ref, ts_ref, out_ref):
    tgt = tgt_ref[...]
    ts = ts_ref[...]
    inv_s = 1.0 / _SCALE
    gx = tgt[:, :, 0] * inv_s
    gy = tgt[:, :, 1] * inv_s
    gh = tgt[:, :, 3] * inv_s
    gw = tgt[:, :, 4] * inv_s

    tt = lax.broadcasted_iota(jnp.int32, (_B, _MAXT), 1)
    valid = (tt < ts) & (gw != 0.0) & (gh != 0.0)

    gi = jnp.clip(gx.astype(jnp.int32), 0, _NW - 1)
    gj = jnp.clip(gy.astype(jnp.int32), 0, _NH - 1)

    a1 = (gw + 1.0) * (gh + 1.0)
    ious = []
    for aw, ah in _ANCHORS:
        aw, ah = aw / _SCALE, ah / _SCALE
        inter = (jnp.clip(jnp.minimum(gw, aw) + 1.0, 0.0, None) *
                 jnp.clip(jnp.minimum(gh, ah) + 1.0, 0.0, None))
        a2 = (aw + 1.0) * (ah + 1.0)
        ious.append(inter / (a1 + a2 - inter + 1e-16))

    best_iou = ious[0]
    best_n = jnp.zeros((_B, _MAXT), jnp.int32)
    for a in range(1, _NA):
        upd = ious[a] > best_iou
        best_n = jnp.where(upd, a, best_n)
        best_iou = jnp.where(upd, ious[a], best_iou)

    validf = valid.astype(jnp.float32)
    ign = [((iou_a > _IGNORE_THRESH) & valid).astype(jnp.float32)
           for iou_a in ious]

    aw_best = jnp.full((_B, _MAXT), _ANCHORS[0][0] / _SCALE)
    ah_best = jnp.full((_B, _MAXT), _ANCHORS[0][1] / _SCALE)
    for a in range(1, _NA):
        sel = best_n == a
        aw_best = jnp.where(sel, _ANCHORS[a][0] / _SCALE, aw_best)
        ah_best = jnp.where(sel, _ANCHORS[a][1] / _SCALE, ah_best)

    def inv_tanh(y):
        yc = jnp.clip(y, -0.999999, 0.999999)
        inner = 0.5 * jnp.log((1.0 + yc) / (1.0 - yc))
        return jnp.where(y <= -1.0, -2.0, jnp.where(y >= 1.0, 2.0, inner))

    txv = inv_tanh(gx - (gi.astype(jnp.float32) + 0.5))
    tyv = inv_tanh(gy - (gj.astype(jnp.float32) + 0.5))
    twv = jnp.log(gw / aw_best + 1e-16)
    thv = jnp.log(gh / ah_best + 1e-16)

    cidx = lax.broadcasted_iota(
        jnp.int32, (_B, _MAXT, _NUM_CLASSES), 2).astype(jnp.float32)
    label = jnp.sum(tgt[:, :, 13:13 + _NUM_CLASSES] * cidx, axis=2)

    cell = (gj * _NW + gi).astype(jnp.float32)
    key1 = (best_n * (_NSEC * _CELLS)).astype(jnp.float32) + cell

    zeros = jnp.zeros((_B, _MAXT), jnp.float32)
    fields = [validf, txv, tyv, twv, thv, label,
              ign[0], ign[1], ign[2], ign[3], ign[4],
              key1, cell, zeros, zeros, zeros]
    for k, f in enumerate(fields):
        out_ref[k] = f


def _make_records(target, target_sizes):
    return pl.pallas_call(
        _records_body,
        out_shape=jax.ShapeDtypeStruct((_NFIELD, _B, _MAXT), jnp.float32),
    )(target, target_sizes.astype(jnp.int32).reshape(_B, 1))


def _sc_scatter_body(rec_hbm, out_hbm, rec_v, comb_v):
    cid = lax.axis_index("c")
    sid = lax.axis_index("s")

    @pl.when(cid == 0)
    def _():
        b = sid
        pltpu.sync_copy(rec_hbm.at[b], rec_v)

        def zero_body(i, carry):
            comb_v[pl.ds(i * 16, 16)] = jnp.zeros((16,), jnp.float32)
            return carry

        lax.fori_loop(0, _COMB // 16, zero_body, 0)

        lane = lax.iota(jnp.int32, 16)

        def tgt_body(t, carry):
            v = plsc.load_gather(rec_v, [lane * _MAXT + t])
            valid = v[0]
            key1 = v[11].astype(jnp.int32)
            cell = v[12].astype(jnp.int32)
            validv = jnp.full((16,), valid) > 0.0
            m1 = (lane < 6) & validv
            idx1 = jnp.where(m1, lane * _CELLS + key1, 0)
            plsc.store_scatter(comb_v, [idx1], v, mask=m1)
            m2 = (lane >= 6) & (lane < 11) & (v > 0.0)
            idx2 = jnp.where(
                m2, (lane - 6) * (_NSEC * _CELLS) + 6 * _CELLS + cell, 0)
            plsc.store_scatter(comb_v, [idx2], v, mask=m2)
            return carry

        lax.fori_loop(0, _MAXT, tgt_body, 0)

        pltpu.sync_copy(comb_v, out_hbm.at[b])


def _sc_scatter(rec):
    mesh = plsc.VectorSubcoreMesh(core_axis_name="c", subcore_axis_name="s",
                                  num_cores=2, num_subcores=16)
    fn = functools.partial(
        pl.kernel,
        out_type=jax.ShapeDtypeStruct((_B, _COMB), jnp.float32),
        mesh=mesh,
        scratch_types=[
            pltpu.VMEM((_REC_FLAT,), jnp.float32),
            pltpu.VMEM((_COMB,), jnp.float32),
        ],
        compiler_params=pltpu.CompilerParams(needs_layout_passes=False),
    )(_sc_scatter_body)
    return fn(rec)


_NROW = _B * _NA


def _loss_body(pred_ref, grids_ref, out_ref, acc_ref):
    i = pl.program_id(0)

    @pl.when(i == 0)
    def _():
        for k in range(9):
            acc_ref[k] = 0.0

    p = pred_ref[0]
    g = grids_ref[0]
    mask = g[0]
    txg = g[1]
    tyg = g[2]
    twg = g[3]
    thg = g[4]
    labg = g[5]
    igng = g[6]

    conf = p[:, 0]
    x = p[:, 1]
    y = p[:, 2]
    h = p[:, 4]
    w = p[:, 5]
    cls = p[:, 6:6 + _NUM_CLASSES]

    ff = (1.0 - mask) * (1.0 - igng)

    bce = (jnp.maximum(conf, 0.0) - conf * mask +
           jnp.log1p(jnp.exp(-jnp.abs(conf))))

    cmax = jnp.max(cls, axis=1)
    lse = cmax + jnp.log(jnp.sum(jnp.exp(cls - cmax[:, None]), axis=1))
    cidx = lax.broadcasted_iota(jnp.int32, (_CELLS, _NUM_CLASSES), 1)
    onehot = (cidx == labg.astype(jnp.int32)[:, None]).astype(jnp.float32)
    picked = jnp.sum(cls * onehot, axis=1) - lse

    acc_ref[0] = acc_ref[0] + jnp.sum(mask)
    acc_ref[1] = acc_ref[1] + jnp.sum(ff)
    acc_ref[2] = acc_ref[2] + jnp.sum(mask * (x - txg) ** 2)
    acc_ref[3] = acc_ref[3] + jnp.sum(mask * (y - tyg) ** 2)
    acc_ref[4] = acc_ref[4] + jnp.sum(mask * (w - twg) ** 2)
    acc_ref[5] = acc_ref[5] + jnp.sum(mask * (h - thg) ** 2)
    acc_ref[6] = acc_ref[6] + jnp.sum(ff * bce)
    acc_ref[7] = acc_ref[7] + jnp.sum(mask * bce)
    acc_ref[8] = acc_ref[8] + jnp.sum(mask * picked)

    @pl.when(i == _NROW - 1)
    def _():
        nM = acc_ref[0]
        nF = acc_ref[1]
        total = ((acc_ref[2] + acc_ref[3] + acc_ref[4] + acc_ref[5]) / nM +
                 _BAD_CONF_WEIGHT * acc_ref[6] / nF + acc_ref[7] / nM -
                 acc_ref[8] / nM)
        out_ref[...] = jnp.full((1, 1), total, jnp.float32)


def _dense_loss(pred_rows, grid_rows):
    return pl.pallas_call(
        _loss_body,
        grid=(_NROW,),
        in_specs=[
            pl.BlockSpec((1, _CELLS, 6 + _NUM_CLASSES), lambda i: (i, 0, 0)),
            pl.BlockSpec((1, _NSEC, _CELLS), lambda i: (i, 0, 0)),
        ],
        out_specs=pl.BlockSpec((1, 1), lambda i: (0, 0)),
        out_shape=jax.ShapeDtypeStruct((1, 1), jnp.float32),
        scratch_shapes=[pltpu.SMEM((16,), jnp.float32)],
    )(pred_rows, grid_rows)


def kernel(prediction, target, target_sizes):
    rec = _make_records(target.astype(jnp.float32), target_sizes)
    rec_b = rec.transpose(1, 0, 2).reshape(_B, _REC_FLAT)
    grids = _sc_scatter(rec_b)
    pred_rows = prediction.reshape(_NROW, _CELLS, 6 + _NUM_CLASSES)
    grid_rows = grids.reshape(_NROW, _NSEC, _CELLS)
    out = _dense_loss(pred_rows, grid_rows)
    return out[0, 0]


# speedup vs baseline: 137.9612x; 1.0294x over previous
import functools

import jax
import jax.numpy as jnp
from jax import lax
from jax.experimental import pallas as pl
from jax.experimental.pallas import tpu as pltpu
from jax.experimental.pallas import tpu_sc as plsc

_NUM_CLASSES = 20
_SCALE = 16.0
_IGNORE_THRESH = 0.5
_BAD_CONF_WEIGHT = 1.25
_ANCHORS = [(25.0, 50.0), (50.0, 100.0), (100.0, 200.0), (200.0, 120.0),
            (320.0, 320.0)]

_B, _NA, _NH, _NW, _MAXT = 16, 5, 32, 32, 50
_CELLS = _NH * _NW
_GRID = _NA * _CELLS
_NFIELD = 16
_NSEC = 7
_REC_FLAT = _NFIELD * _MAXT
_COMB = _NSEC * _GRID


def _records_body(tgt_ref, ts_ref, out_ref):
    tgt = tgt_ref[...]
    ts = ts_ref[...]
    inv_s = 1.0 / _SCALE
    gx = tgt[:, :, 0] * inv_s
    gy = tgt[:, :, 1] * inv_s
    gh = tgt[:, :, 3] * inv_s
    gw = tgt[:, :, 4] * inv_s

    tt = lax.broadcasted_iota(jnp.int32, (_B, _MAXT), 1)
    valid = (tt < ts) & (gw != 0.0) & (gh != 0.0)

    gi = jnp.clip(gx.astype(jnp.int32), 0, _NW - 1)
    gj = jnp.clip(gy.astype(jnp.int32), 0, _NH - 1)

    a1 = (gw + 1.0) * (gh + 1.0)
    ious = []
    for aw, ah in _ANCHORS:
        aw, ah = aw / _SCALE, ah / _SCALE
        inter = (jnp.clip(jnp.minimum(gw, aw) + 1.0, 0.0, None) *
                 jnp.clip(jnp.minimum(gh, ah) + 1.0, 0.0, None))
        a2 = (aw + 1.0) * (ah + 1.0)
        ious.append(inter / (a1 + a2 - inter + 1e-16))

    best_iou = ious[0]
    best_n = jnp.zeros((_B, _MAXT), jnp.int32)
    for a in range(1, _NA):
        upd = ious[a] > best_iou
        best_n = jnp.where(upd, a, best_n)
        best_iou = jnp.where(upd, ious[a], best_iou)

    validf = valid.astype(jnp.float32)
    ign = [((iou_a > _IGNORE_THRESH) & valid).astype(jnp.float32)
           for iou_a in ious]

    aw_best = jnp.full((_B, _MAXT), _ANCHORS[0][0] / _SCALE)
    ah_best = jnp.full((_B, _MAXT), _ANCHORS[0][1] / _SCALE)
    for a in range(1, _NA):
        sel = best_n == a
        aw_best = jnp.where(sel, _ANCHORS[a][0] / _SCALE, aw_best)
        ah_best = jnp.where(sel, _ANCHORS[a][1] / _SCALE, ah_best)

    def inv_tanh(y):
        yc = jnp.clip(y, -0.999999, 0.999999)
        inner = 0.5 * jnp.log((1.0 + yc) / (1.0 - yc))
        return jnp.where(y <= -1.0, -2.0, jnp.where(y >= 1.0, 2.0, inner))

    txv = inv_tanh(gx - (gi.astype(jnp.float32) + 0.5))
    tyv = inv_tanh(gy - (gj.astype(jnp.float32) + 0.5))
    twv = jnp.log(gw / aw_best + 1e-16)
    thv = jnp.log(gh / ah_best + 1e-16)

    cidx = lax.broadcasted_iota(
        jnp.int32, (_B, _MAXT, _NUM_CLASSES), 2).astype(jnp.float32)
    label = jnp.sum(tgt[:, :, 13:13 + _NUM_CLASSES] * cidx, axis=2)

    cell = (gj * _NW + gi).astype(jnp.float32)
    key1 = (best_n * (_NSEC * _CELLS)).astype(jnp.float32) + cell

    zeros = jnp.zeros((_B, _MAXT), jnp.float32)
    fields = [validf, txv, tyv, twv, thv, label,
              ign[0], ign[1], ign[2], ign[3], ign[4],
              key1, cell, zeros, zeros, zeros]
    for k, f in enumerate(fields):
        out_ref[k] = f


def _make_records(target, target_sizes):
    return pl.pallas_call(
        _records_body,
        out_shape=jax.ShapeDtypeStruct((_NFIELD, _B, _MAXT), jnp.float32),
    )(target, target_sizes.astype(jnp.int32).reshape(_B, 1))


def _sc_scatter_body(rec_hbm, out_hbm, rec_v, comb_v):
    cid = lax.axis_index("c")
    sid = lax.axis_index("s")

    @pl.when(cid == 0)
    def _():
        b = sid
        pltpu.sync_copy(rec_hbm.at[:, b], rec_v)

        def zero_body(i, carry):
            zero = jnp.zeros((16,), jnp.float32)
            for u in range(8):
                comb_v[pl.ds(i * 128 + u * 16, 16)] = zero
            return carry

        lax.fori_loop(0, _COMB // 128, zero_body, 0)

        lane = lax.iota(jnp.int32, 16)

        def tgt_body(t, carry):
            v = plsc.load_gather(rec_v, [lane, jnp.full((16,), t, jnp.int32)])
            valid = v[0]
            key1 = v[11].astype(jnp.int32)
            cell = v[12].astype(jnp.int32)
            validv = jnp.full((16,), valid) > 0.0
            m1 = (lane < 6) & validv
            idx1 = jnp.where(m1, lane * _CELLS + key1, 0)
            plsc.store_scatter(comb_v, [idx1], v, mask=m1)
            m2 = (lane >= 6) & (lane < 11) & (v > 0.0)
            idx2 = jnp.where(
                m2, (lane - 6) * (_NSEC * _CELLS) + 6 * _CELLS + cell, 0)
            plsc.store_scatter(comb_v, [idx2], v, mask=m2)
            return carry

        lax.fori_loop(0, _MAXT, tgt_body, 0)

        pltpu.sync_copy(comb_v, out_hbm.at[b])


def _sc_scatter(rec):
    mesh = plsc.VectorSubcoreMesh(core_axis_name="c", subcore_axis_name="s",
                                  num_cores=2, num_subcores=16)
    fn = functools.partial(
        pl.kernel,
        out_type=jax.ShapeDtypeStruct((_B, _COMB), jnp.float32),
        mesh=mesh,
        scratch_types=[
            pltpu.VMEM((_NFIELD, _MAXT), jnp.float32),
            pltpu.VMEM((_COMB,), jnp.float32),
        ],
        compiler_params=pltpu.CompilerParams(needs_layout_passes=False),
    )(_sc_scatter_body)
    return fn(rec)


_NROW = _B * _NA


def _loss_body(pred_ref, grids_ref, out_ref, acc_ref):
    i = pl.program_id(0)

    @pl.when(i == 0)
    def _():
        for k in range(9):
            acc_ref[k] = 0.0

    p = pred_ref[0]
    g = grids_ref[0]
    mask = g[0]
    txg = g[1]
    tyg = g[2]
    twg = g[3]
    thg = g[4]
    labg = g[5]
    igng = g[6]

    conf = p[:, 0]
    x = p[:, 1]
    y = p[:, 2]
    h = p[:, 4]
    w = p[:, 5]
    cls = p[:, 6:6 + _NUM_CLASSES]

    ff = (1.0 - mask) * (1.0 - igng)

    bce = (jnp.maximum(conf, 0.0) - conf * mask +
           jnp.log1p(jnp.exp(-jnp.abs(conf))))

    cmax = jnp.max(cls, axis=1)
    lse = cmax + jnp.log(jnp.sum(jnp.exp(cls - cmax[:, None]), axis=1))
    cidx = lax.broadcasted_iota(jnp.int32, (_CELLS, _NUM_CLASSES), 1)
    onehot = (cidx == labg.astype(jnp.int32)[:, None]).astype(jnp.float32)
    picked = jnp.sum(cls * onehot, axis=1) - lse

    acc_ref[0] = acc_ref[0] + jnp.sum(mask)
    acc_ref[1] = acc_ref[1] + jnp.sum(ff)
    acc_ref[2] = acc_ref[2] + jnp.sum(mask * (x - txg) ** 2)
    acc_ref[3] = acc_ref[3] + jnp.sum(mask * (y - tyg) ** 2)
    acc_ref[4] = acc_ref[4] + jnp.sum(mask * (w - twg) ** 2)
    acc_ref[5] = acc_ref[5] + jnp.sum(mask * (h - thg) ** 2)
    acc_ref[6] = acc_ref[6] + jnp.sum(ff * bce)
    acc_ref[7] = acc_ref[7] + jnp.sum(mask * bce)
    acc_ref[8] = acc_ref[8] + jnp.sum(mask * picked)

    @pl.when(i == _NROW - 1)
    def _():
        nM = acc_ref[0]
        nF = acc_ref[1]
        total = ((acc_ref[2] + acc_ref[3] + acc_ref[4] + acc_ref[5]) / nM +
                 _BAD_CONF_WEIGHT * acc_ref[6] / nF + acc_ref[7] / nM -
                 acc_ref[8] / nM)
        out_ref[...] = jnp.full((1, 1), total, jnp.float32)


def _dense_loss(pred_rows, grid_rows):
    return pl.pallas_call(
        _loss_body,
        grid=(_NROW,),
        in_specs=[
            pl.BlockSpec((1, _CELLS, 6 + _NUM_CLASSES), lambda i: (i, 0, 0)),
            pl.BlockSpec((1, _NSEC, _CELLS), lambda i: (i, 0, 0)),
        ],
        out_specs=pl.BlockSpec((1, 1), lambda i: (0, 0)),
        out_shape=jax.ShapeDtypeStruct((1, 1), jnp.float32),
        scratch_shapes=[pltpu.SMEM((16,), jnp.float32)],
    )(pred_rows, grid_rows)


def kernel(prediction, target, target_sizes):
    rec = _make_records(target.astype(jnp.float32), target_sizes)
    grids = _sc_scatter(rec)
    pred_rows = prediction.reshape(_NROW, _CELLS, 6 + _NUM_CLASSES)
    grid_rows = grids.reshape(_NROW, _NSEC, _CELLS)
    out = _dense_loss(pred_rows, grid_rows)
    return out[0, 0]


# speedup vs baseline: 686.5049x; 4.9761x over previous
import functools

import jax
import jax.numpy as jnp
from jax import lax
from jax.experimental import pallas as pl
from jax.experimental.pallas import tpu as pltpu
from jax.experimental.pallas import tpu_sc as plsc

_NUM_CLASSES = 20
_SCALE = 16.0
_IGNORE_THRESH = 0.5
_BAD_CONF_WEIGHT = 1.25
_ANCHORS = [(25.0, 50.0), (50.0, 100.0), (100.0, 200.0), (200.0, 120.0),
            (320.0, 320.0)]

_B, _NA, _NH, _NW, _MAXT = 16, 5, 32, 32, 50
_CELLS = _NH * _NW
_GRID = _NA * _CELLS
_NFIELD = 16
_NSEC = 7
_REC_FLAT = _NFIELD * _MAXT
_COMB = _NSEC * _GRID


def _records_body(tgt_ref, ts_ref, out_ref):
    tgt = tgt_ref[...]
    ts = ts_ref[...]
    inv_s = 1.0 / _SCALE
    gx = tgt[:, :, 0] * inv_s
    gy = tgt[:, :, 1] * inv_s
    gh = tgt[:, :, 3] * inv_s
    gw = tgt[:, :, 4] * inv_s

    tt = lax.broadcasted_iota(jnp.int32, (_B, _MAXT), 1)
    valid = (tt < ts) & (gw != 0.0) & (gh != 0.0)

    gi = jnp.clip(gx.astype(jnp.int32), 0, _NW - 1)
    gj = jnp.clip(gy.astype(jnp.int32), 0, _NH - 1)

    a1 = (gw + 1.0) * (gh + 1.0)
    ious = []
    for aw, ah in _ANCHORS:
        aw, ah = aw / _SCALE, ah / _SCALE
        inter = (jnp.clip(jnp.minimum(gw, aw) + 1.0, 0.0, None) *
                 jnp.clip(jnp.minimum(gh, ah) + 1.0, 0.0, None))
        a2 = (aw + 1.0) * (ah + 1.0)
        ious.append(inter / (a1 + a2 - inter + 1e-16))

    best_iou = ious[0]
    best_n = jnp.zeros((_B, _MAXT), jnp.int32)
    for a in range(1, _NA):
        upd = ious[a] > best_iou
        best_n = jnp.where(upd, a, best_n)
        best_iou = jnp.where(upd, ious[a], best_iou)

    validf = valid.astype(jnp.float32)
    ign = [((iou_a > _IGNORE_THRESH) & valid).astype(jnp.float32)
           for iou_a in ious]

    aw_best = jnp.full((_B, _MAXT), _ANCHORS[0][0] / _SCALE)
    ah_best = jnp.full((_B, _MAXT), _ANCHORS[0][1] / _SCALE)
    for a in range(1, _NA):
        sel = best_n == a
        aw_best = jnp.where(sel, _ANCHORS[a][0] / _SCALE, aw_best)
        ah_best = jnp.where(sel, _ANCHORS[a][1] / _SCALE, ah_best)

    def inv_tanh(y):
        yc = jnp.clip(y, -0.999999, 0.999999)
        inner = 0.5 * jnp.log((1.0 + yc) / (1.0 - yc))
        return jnp.where(y <= -1.0, -2.0, jnp.where(y >= 1.0, 2.0, inner))

    txv = inv_tanh(gx - (gi.astype(jnp.float32) + 0.5))
    tyv = inv_tanh(gy - (gj.astype(jnp.float32) + 0.5))
    twv = jnp.log(gw / aw_best + 1e-16)
    thv = jnp.log(gh / ah_best + 1e-16)

    cidx = lax.broadcasted_iota(
        jnp.int32, (_B, _MAXT, _NUM_CLASSES), 2).astype(jnp.float32)
    label = jnp.sum(tgt[:, :, 13:13 + _NUM_CLASSES] * cidx, axis=2)

    cell = (gj * _NW + gi).astype(jnp.float32)
    key1 = (best_n * (_NSEC * _CELLS)).astype(jnp.float32) + cell

    zeros = jnp.zeros((_B, _MAXT), jnp.float32)
    fields = [validf, txv, tyv, twv, thv, label,
              ign[0], ign[1], ign[2], ign[3], ign[4],
              key1, cell, zeros, zeros, zeros]
    for k, f in enumerate(fields):
        out_ref[k] = f


def _make_records(target, target_sizes):
    return pl.pallas_call(
        _records_body,
        out_shape=jax.ShapeDtypeStruct((_NFIELD, _B, _MAXT), jnp.float32),
    )(target, target_sizes.astype(jnp.int32).reshape(_B, 1))


def _sc_scatter_body(rec_hbm, out_hbm, rec_v, comb_v):
    cid = lax.axis_index("c")
    sid = lax.axis_index("s")

    @pl.when(cid == 0)
    def _():
        b = sid
        pltpu.sync_copy(rec_hbm.at[:, b], rec_v)

        def zero_body(i, carry):
            zero = jnp.zeros((16,), jnp.float32)
            for u in range(8):
                comb_v[pl.ds(i * 128 + u * 16, 16)] = zero
            return carry

        lax.fori_loop(0, _COMB // 128, zero_body, 0)

        lane = lax.iota(jnp.int32, 16)

        def tgt_body(t, carry):
            v = plsc.load_gather(rec_v, [lane, jnp.full((16,), t, jnp.int32)])
            valid = v[0]
            key1 = v[11].astype(jnp.int32)
            cell = v[12].astype(jnp.int32)
            validv = jnp.full((16,), valid) > 0.0
            m1 = (lane < 6) & validv
            idx1 = jnp.where(m1, lane * _CELLS + key1, 0)
            plsc.store_scatter(comb_v, [idx1], v, mask=m1)
            m2 = (lane >= 6) & (lane < 11) & (v > 0.0)
            idx2 = jnp.where(
                m2, (lane - 6) * (_NSEC * _CELLS) + 6 * _CELLS + cell, 0)
            plsc.store_scatter(comb_v, [idx2], v, mask=m2)
            return carry

        lax.fori_loop(0, _MAXT, tgt_body, 0)

        pltpu.sync_copy(comb_v, out_hbm.at[pl.ds(b * _COMB, _COMB)])


def _sc_scatter(rec):
    mesh = plsc.VectorSubcoreMesh(core_axis_name="c", subcore_axis_name="s",
                                  num_cores=2, num_subcores=16)
    fn = functools.partial(
        pl.kernel,
        out_type=jax.ShapeDtypeStruct((_B * _COMB,), jnp.float32),
        mesh=mesh,
        scratch_types=[
            pltpu.VMEM((_NFIELD, _MAXT), jnp.float32),
            pltpu.VMEM((_COMB,), jnp.float32),
        ],
        compiler_params=pltpu.CompilerParams(needs_layout_passes=False),
    )(_sc_scatter_body)
    return fn(rec)


_NROW = _B * _NA
_RSTEP = 8
_NSTEP = _NROW // _RSTEP


def _loss_body(pred_ref, grids_ref, out_ref, acc_ref):
    i = pl.program_id(0)

    @pl.when(i == 0)
    def _():
        acc_ref[...] = jnp.zeros_like(acc_ref)

    p = pred_ref[...]
    g = grids_ref[...]
    mask = g[:, 0]
    txg = g[:, 1]
    tyg = g[:, 2]
    twg = g[:, 3]
    thg = g[:, 4]
    labg = g[:, 5]
    igng = g[:, 6]

    conf = p[:, 0]
    x = p[:, 1]
    y = p[:, 2]
    h = p[:, 4]
    w = p[:, 5]

    ff = (1.0 - mask) * (1.0 - igng)

    bce = (jnp.maximum(conf, 0.0) - conf * mask +
           jnp.log1p(jnp.exp(-jnp.abs(conf))))

    cmax = p[:, 6]
    for c in range(1, _NUM_CLASSES):
        cmax = jnp.maximum(cmax, p[:, 6 + c])
    esum = jnp.zeros_like(cmax)
    picked = jnp.zeros_like(cmax)
    labi = labg.astype(jnp.int32)
    for c in range(_NUM_CLASSES):
        cls_c = p[:, 6 + c]
        esum = esum + jnp.exp(cls_c - cmax)
        picked = picked + jnp.where(labi == c, cls_c, 0.0)
    lse = cmax + jnp.log(esum)
    picked = picked - lse

    def rsum(v):
        return jnp.sum(v, axis=0)

    acc = acc_ref[...]
    terms = [mask,
             ff,
             mask * (x - txg) ** 2,
             mask * (y - tyg) ** 2,
             mask * (w - twg) ** 2,
             mask * (h - thg) ** 2,
             ff * bce,
             mask * bce,
             mask * picked]
    acc_ref[...] = acc + jnp.stack([rsum(t) for t in terms], axis=0)

    @pl.when(i == _NSTEP - 1)
    def _():
        s = jnp.sum(acc_ref[...], axis=(1, 2))
        nM = s[0]
        nF = s[1]
        total = ((s[2] + s[3] + s[4] + s[5]) / nM +
                 _BAD_CONF_WEIGHT * s[6] / nF + s[7] / nM - s[8] / nM)
        out_ref[...] = jnp.full((1, 1), total, jnp.float32)


def _dense_loss(pred_rows, grid_rows):
    return pl.pallas_call(
        _loss_body,
        grid=(_NSTEP,),
        in_specs=[
            pl.BlockSpec((_RSTEP, 6 + _NUM_CLASSES, 8, 128),
                         lambda i: (i, 0, 0, 0)),
            pl.BlockSpec((_RSTEP, _NSEC, 8, 128), lambda i: (i, 0, 0, 0)),
        ],
        out_specs=pl.BlockSpec((1, 1), lambda i: (0, 0)),
        out_shape=jax.ShapeDtypeStruct((1, 1), jnp.float32),
        scratch_shapes=[pltpu.VMEM((9, 8, 128), jnp.float32)],
    )(pred_rows, grid_rows)


def kernel(prediction, target, target_sizes):
    rec = _make_records(target.astype(jnp.float32), target_sizes)
    grids = _sc_scatter(rec)
    pred_rows = prediction.transpose(0, 1, 4, 2, 3).reshape(
        _NROW, 6 + _NUM_CLASSES, 8, _NH * _NW // 8)
    grid_rows = grids.reshape(_NROW, _NSEC, 8, _NH * _NW // 8)
    out = _dense_loss(pred_rows, grid_rows)
    return out[0, 0]
